# baseline (speedup 1.0000x reference)
import numpy as np

import jax
import jax.numpy as jnp
from jax import lax
from jax.experimental import pallas as pl
from jax.experimental.pallas import tpu as pltpu
from jax.experimental.pallas import tpu_sc as plsc

N = 10000
NFEAT = 128
NHID = 64
NCLASS = 16

BM = 1024
NB = 10
NPAD = NB * BM
NT2 = NB * (NB - 1) // 2


LAST = N - (NB - 1) * BM


def _call1_body(adj_ref, x_ref, w1_ref, b1_ref, w2_ref, b2_ref,
                outp_ref, hw_out_ref, xw_scr, hacc_scr, hw_scr, diag_scr):
    p = pl.program_id(0)
    c = pl.program_id(1)

    @pl.when((p == 0) & (c == 0))
    def _():
        xw_scr[...] = jnp.dot(x_ref[...], w1_ref[...],
                              preferred_element_type=jnp.float32)

    @pl.when(c == 0)
    def _():
        hacc_scr[...] = jnp.dot(adj_ref[...], xw_scr[pl.ds(0, BM), :],
                                precision=lax.Precision.DEFAULT,
                                preferred_element_type=jnp.float32)
        outp_ref[...] = jnp.broadcast_to(b2_ref[...], (BM, NCLASS))

    @pl.when((c > 0) & (c < NB - 1))
    def _():
        hacc_scr[...] += jnp.dot(adj_ref[...], xw_scr[pl.ds(c * BM, BM), :],
                                 precision=lax.Precision.DEFAULT,
                                 preferred_element_type=jnp.float32)

    @pl.when(c < p)
    def _():
        outp_ref[...] += jnp.dot(adj_ref[...], hw_scr[pl.ds(c * BM, BM), :],
                                 precision=lax.Precision.DEFAULT,
                                 preferred_element_type=jnp.float32)

    @pl.when((c == p) & (c < NB - 1))
    def _():
        diag_scr[...] = adj_ref[...]

    @pl.when(c == NB - 1)
    def _():
        hacc_scr[...] += jnp.dot(adj_ref[:, 0:LAST],
                                 xw_scr[pl.ds((NB - 1) * BM, LAST), :],
                                 precision=lax.Precision.DEFAULT,
                                 preferred_element_type=jnp.float32)
        h = jnp.maximum(hacc_scr[...] + b1_ref[...], 0.0)
        hwp = jnp.dot(h, w2_ref[...], preferred_element_type=jnp.float32)
        hw_scr[pl.ds(p * BM, BM), :] = hwp
        hw_out_ref[...] = hwp

    @pl.when((c == NB - 1) & (p < NB - 1))
    def _():
        outp_ref[...] += jnp.dot(diag_scr[...], hw_scr[pl.ds(p * BM, BM), :],
                                 precision=lax.Precision.DEFAULT,
                                 preferred_element_type=jnp.float32)

    @pl.when((c == NB - 1) & (p == NB - 1))
    def _():
        outp_ref[...] += jnp.dot(adj_ref[:, 0:LAST],
                                 hw_scr[pl.ds(p * BM, LAST), :],
                                 precision=lax.Precision.DEFAULT,
                                 preferred_element_type=jnp.float32)


_call1 = pl.pallas_call(
    _call1_body,
    grid=(NB, NB),
    in_specs=[
        pl.BlockSpec((BM, BM), lambda p, c: (p, c)),
        pl.BlockSpec((N, NFEAT), lambda p, c: (0, 0)),
        pl.BlockSpec((NFEAT, NHID), lambda p, c: (0, 0)),
        pl.BlockSpec((1, NHID), lambda p, c: (0, 0)),
        pl.BlockSpec((NHID, NCLASS), lambda p, c: (0, 0)),
        pl.BlockSpec((1, NCLASS), lambda p, c: (0, 0)),
    ],
    out_specs=[
        pl.BlockSpec((BM, NCLASS), lambda p, c: (p, 0)),
        pl.BlockSpec((BM, NCLASS), lambda p, c: (p, 0)),
    ],
    out_shape=[
        jax.ShapeDtypeStruct((N, NCLASS), jnp.float32),
        jax.ShapeDtypeStruct((NPAD, NCLASS), jnp.float32),
    ],
    scratch_shapes=[
        pltpu.VMEM((N, NHID), jnp.float32),
        pltpu.VMEM((BM, NHID), jnp.float32),
        pltpu.VMEM((NPAD, NCLASS), jnp.float32),
        pltpu.VMEM((BM, BM), jnp.float32),
    ],
    compiler_params=pltpu.CompilerParams(
        dimension_semantics=("arbitrary", "arbitrary")),
)

_PU = np.concatenate([np.full(NB - 1 - p, p, np.int32) for p in range(NB - 1)])
_CU = np.concatenate([np.arange(p + 1, NB, dtype=np.int32)
                      for p in range(NB - 1)])


def _call2_body(ptab, ctab, adj_ref, hw_ref, outp_ref, out_ref, acc_scr):
    t = pl.program_id(0)
    p = ptab[t]
    c = ctab[t]

    @pl.when(t == 0)
    def _():
        acc_scr[0:N, :] = outp_ref[...]

    @pl.when(c < NB - 1)
    def _():
        acc_scr[pl.ds(p * BM, BM), :] += jnp.dot(
            adj_ref[...], hw_ref[pl.ds(c * BM, BM), :],
            precision=lax.Precision.DEFAULT,
            preferred_element_type=jnp.float32)

    @pl.when(c == NB - 1)
    def _():
        acc_scr[pl.ds(p * BM, BM), :] += jnp.dot(
            adj_ref[:, 0:LAST], hw_ref[pl.ds(c * BM, LAST), :],
            precision=lax.Precision.DEFAULT,
            preferred_element_type=jnp.float32)

    @pl.when(t == NT2 - 1)
    def _():
        out_ref[...] = acc_scr[0:N, :]


_call2 = pl.pallas_call(
    _call2_body,
    grid_spec=pltpu.PrefetchScalarGridSpec(
        num_scalar_prefetch=2,
        grid=(NT2,),
        in_specs=[
            pl.BlockSpec((BM, BM), lambda t, pt, ct: (pt[t], ct[t])),
            pl.BlockSpec((NPAD, NCLASS), lambda t, pt, ct: (0, 0)),
            pl.BlockSpec((N, NCLASS), lambda t, pt, ct: (0, 0)),
        ],
        out_specs=pl.BlockSpec((N, NCLASS), lambda t, pt, ct: (0, 0)),
        scratch_shapes=[pltpu.VMEM((NPAD, NCLASS), jnp.float32)],
    ),
    out_shape=jax.ShapeDtypeStruct((N, NCLASS), jnp.float32),
    compiler_params=pltpu.CompilerParams(
        dimension_semantics=("arbitrary",)),
)


def _perm(t, idx):
    dnums = lax.GatherDimensionNumbers(
        offset_dims=(), collapsed_slice_dims=(0,), start_index_map=(0,))
    return lax.gather(t, idx[:, None], dnums, slice_sizes=(1,),
                      mode=lax.GatherScatterMode.PROMISE_IN_BOUNDS)


def _head_body(edge_hbm, emb_flat_hbm, out_hbm, idx_v, a_v, b_v, sig_v, sem):
    c = lax.axis_index("c")
    s = lax.axis_index("s")

    @pl.when((c == 0) & (s == 0))
    def _():
        pltpu.sync_copy(edge_hbm, idx_v)
        ev = idx_v[...]
        lanes = lax.iota(jnp.int32, 16)
        nd1 = _perm(ev, jnp.zeros((16,), jnp.int32))
        nd2 = _perm(ev, jnp.ones((16,), jnp.int32))
        pltpu.async_copy(emb_flat_hbm.at[nd1 * NCLASS + lanes], a_v, sem).wait()
        pltpu.async_copy(emb_flat_hbm.at[nd2 * NCLASS + lanes], b_v, sem).wait()
        t = a_v[...] * b_v[...]
        for shift in (8, 4, 2, 1):
            t = t + _perm(t, (lanes + shift) % 16)
        sig_v[...] = 1.0 / (1.0 + jnp.exp(-t))
        pltpu.sync_copy(sig_v, out_hbm)


def _make_head():
    return pl.kernel(
        _head_body,
        mesh=plsc.VectorSubcoreMesh(core_axis_name="c", subcore_axis_name="s"),
        out_type=jax.ShapeDtypeStruct((16,), jnp.float32),
        scratch_types=[
            pltpu.VMEM((16,), jnp.int32),
            pltpu.VMEM((16,), jnp.float32),
            pltpu.VMEM((16,), jnp.float32),
            pltpu.VMEM((16,), jnp.float32),
            pltpu.SemaphoreType.DMA,
        ],
    )


def kernel(x, adj, train_edge, train_label, W1, b1, W2, b2):
    outp, hw = _call1(adj, x, W1, b1.reshape(1, NHID),
                      W2, b2.reshape(1, NCLASS))
    out = _call2(jnp.asarray(_PU), jnp.asarray(_CU), adj, hw, outp)
    edge16 = jnp.zeros((16,), jnp.int32).at[:2].set(train_edge.astype(jnp.int32))
    sig16 = _make_head()(edge16, out.reshape(N * NCLASS))
    return (out, sig16[0])


# speedup vs baseline: 1.1970x; 1.1970x over previous
import numpy as np

import jax
import jax.numpy as jnp
from jax import lax
from jax.experimental import pallas as pl
from jax.experimental.pallas import tpu as pltpu
from jax.experimental.pallas import tpu_sc as plsc

N = 10000
NFEAT = 128
NHID = 64
NCLASS = 16
NCOMB = NHID + NCLASS

BM = 1280
NB = 8
NPAD = NB * BM
LAST = N - (NB - 1) * BM
NT2 = NB * (NB - 1) // 2


def _xw_body(x_ref, w1_ref, xw_ref):
    xw_ref[...] = jnp.dot(x_ref[...], w1_ref[...],
                          preferred_element_type=jnp.float32)


_xw_call = pl.pallas_call(
    _xw_body,
    out_shape=jax.ShapeDtypeStruct((N, NHID), jnp.float32),
)


def _call1_body(adj_ref, xw_ref, b1_ref, w2_ref, b2_ref,
                outp_ref, hw_out_ref, pw_scr, hacc_scr, diag_scr):
    p = pl.program_id(0)
    c = pl.program_id(1)

    @pl.when((p == 0) & (c == 0))
    def _():
        pw_scr[0:N, 0:NHID] = xw_ref[...]

    @pl.when(c < NB - 1)
    def _():
        part = jnp.dot(adj_ref[...], pw_scr[pl.ds(c * BM, BM), :],
                       preferred_element_type=jnp.float32)

        @pl.when(c == 0)
        def _():
            hacc_scr[...] = part
            outp_ref[...] = jnp.broadcast_to(b2_ref[...], (BM, NCLASS))

        @pl.when(c > 0)
        def _():
            hacc_scr[...] += part

        @pl.when(c < p)
        def _():
            outp_ref[...] += part[:, NHID:NCOMB]

        @pl.when(c == p)
        def _():
            diag_scr[...] = adj_ref[...]

    @pl.when(c == NB - 1)
    def _():
        hacc_scr[...] += jnp.dot(adj_ref[:, 0:LAST],
                                 pw_scr[pl.ds((NB - 1) * BM, LAST), :],
                                 preferred_element_type=jnp.float32)
        h = jnp.maximum(hacc_scr[:, 0:NHID] + b1_ref[...], 0.0)
        hwp = jnp.dot(h, w2_ref[...], preferred_element_type=jnp.float32)
        pw_scr[pl.ds(p * BM, BM), NHID:NCOMB] = hwp
        hw_out_ref[...] = hwp

        @pl.when(p < NB - 1)
        def _():
            outp_ref[...] += jnp.dot(diag_scr[...], hwp,
                                     preferred_element_type=jnp.float32)

        @pl.when(p == NB - 1)
        def _():
            outp_ref[...] += jnp.dot(adj_ref[:, 0:LAST], hwp[0:LAST, :],
                                     preferred_element_type=jnp.float32)


_call1 = pl.pallas_call(
    _call1_body,
    grid=(NB, NB),
    in_specs=[
        pl.BlockSpec((BM, BM), lambda p, c: (p, c)),
        pl.BlockSpec((N, NHID), lambda p, c: (0, 0)),
        pl.BlockSpec((1, NHID), lambda p, c: (0, 0)),
        pl.BlockSpec((NHID, NCLASS), lambda p, c: (0, 0)),
        pl.BlockSpec((1, NCLASS), lambda p, c: (0, 0)),
    ],
    out_specs=[
        pl.BlockSpec((BM, NCLASS), lambda p, c: (p, 0)),
        pl.BlockSpec((BM, NCLASS), lambda p, c: (p, 0)),
    ],
    out_shape=[
        jax.ShapeDtypeStruct((N, NCLASS), jnp.float32),
        jax.ShapeDtypeStruct((NPAD, NCLASS), jnp.float32),
    ],
    scratch_shapes=[
        pltpu.VMEM((NPAD, NCOMB), jnp.float32),
        pltpu.VMEM((BM, NCOMB), jnp.float32),
        pltpu.VMEM((BM, BM), jnp.float32),
    ],
    compiler_params=pltpu.CompilerParams(
        dimension_semantics=("arbitrary", "arbitrary")),
)

_PU = np.concatenate([np.full(NB - 1 - p, p, np.int32) for p in range(NB - 1)])
_CU = np.concatenate([np.arange(p + 1, NB, dtype=np.int32)
                      for p in range(NB - 1)])


def _call2_body(ptab, ctab, adj_ref, hw_ref, outp_ref, out_ref, acc_scr):
    t = pl.program_id(0)
    p = ptab[t]
    c = ctab[t]

    @pl.when(t == 0)
    def _():
        acc_scr[0:N, :] = outp_ref[...]

    @pl.when(c < NB - 1)
    def _():
        acc_scr[pl.ds(p * BM, BM), :] += jnp.dot(
            adj_ref[...], hw_ref[pl.ds(c * BM, BM), :],
            preferred_element_type=jnp.float32)

    @pl.when(c == NB - 1)
    def _():
        acc_scr[pl.ds(p * BM, BM), :] += jnp.dot(
            adj_ref[:, 0:LAST], hw_ref[pl.ds(c * BM, LAST), :],
            preferred_element_type=jnp.float32)

    @pl.when(t == NT2 - 1)
    def _():
        out_ref[...] = acc_scr[0:N, :]


_call2 = pl.pallas_call(
    _call2_body,
    grid_spec=pltpu.PrefetchScalarGridSpec(
        num_scalar_prefetch=2,
        grid=(NT2,),
        in_specs=[
            pl.BlockSpec((BM, BM), lambda t, pt, ct: (pt[t], ct[t])),
            pl.BlockSpec((NPAD, NCLASS), lambda t, pt, ct: (0, 0)),
            pl.BlockSpec((N, NCLASS), lambda t, pt, ct: (0, 0)),
        ],
        out_specs=pl.BlockSpec((N, NCLASS), lambda t, pt, ct: (0, 0)),
        scratch_shapes=[pltpu.VMEM((NPAD, NCLASS), jnp.float32)],
    ),
    out_shape=jax.ShapeDtypeStruct((N, NCLASS), jnp.float32),
    compiler_params=pltpu.CompilerParams(
        dimension_semantics=("arbitrary",)),
)


def _perm(t, idx):
    dnums = lax.GatherDimensionNumbers(
        offset_dims=(), collapsed_slice_dims=(0,), start_index_map=(0,))
    return lax.gather(t, idx[:, None], dnums, slice_sizes=(1,),
                      mode=lax.GatherScatterMode.PROMISE_IN_BOUNDS)


def _head_body(edge_hbm, emb_flat_hbm, out_hbm, idx_v, a_v, b_v, sig_v, sem):
    c = lax.axis_index("c")
    s = lax.axis_index("s")

    @pl.when((c == 0) & (s == 0))
    def _():
        pltpu.sync_copy(edge_hbm, idx_v)
        ev = idx_v[...]
        lanes = lax.iota(jnp.int32, 16)
        nd1 = _perm(ev, jnp.zeros((16,), jnp.int32))
        nd2 = _perm(ev, jnp.ones((16,), jnp.int32))
        pltpu.async_copy(emb_flat_hbm.at[nd1 * NCLASS + lanes], a_v, sem).wait()
        pltpu.async_copy(emb_flat_hbm.at[nd2 * NCLASS + lanes], b_v, sem).wait()
        t = a_v[...] * b_v[...]
        for shift in (8, 4, 2, 1):
            t = t + _perm(t, (lanes + shift) % 16)
        sig_v[...] = 1.0 / (1.0 + jnp.exp(-t))
        pltpu.sync_copy(sig_v, out_hbm)


def _make_head():
    return pl.kernel(
        _head_body,
        mesh=plsc.VectorSubcoreMesh(core_axis_name="c", subcore_axis_name="s"),
        out_type=jax.ShapeDtypeStruct((16,), jnp.float32),
        scratch_types=[
            pltpu.VMEM((16,), jnp.int32),
            pltpu.VMEM((16,), jnp.float32),
            pltpu.VMEM((16,), jnp.float32),
            pltpu.VMEM((16,), jnp.float32),
            pltpu.SemaphoreType.DMA,
        ],
    )


def kernel(x, adj, train_edge, train_label, W1, b1, W2, b2):
    xw = _xw_call(x, W1)
    outp, hw = _call1(adj, xw, b1.reshape(1, NHID),
                      W2, b2.reshape(1, NCLASS))
    out = _call2(jnp.asarray(_PU), jnp.asarray(_CU), adj, hw, outp)
    edge16 = jnp.zeros((16,), jnp.int32).at[:2].set(train_edge.astype(jnp.int32))
    sig16 = _make_head()(edge16, out.reshape(N * NCLASS))
    return (out, sig16[0])
